# baseline (speedup 1.0000x reference)
import functools

import numpy as np
import jax
import jax.numpy as jnp
from jax import lax
from jax.experimental import pallas as pl
from jax.experimental.pallas import tpu as pltpu

_EPS = 1e-6
_HB = 64
_INSZ = 256
_SRC = 512
_R_CHUNK = 32
_N_CHUNK = _INSZ // _R_CHUNK
_NP = _R_CHUNK * _INSZ


def _resize_weight_mat(insz: int, outsz: int) -> np.ndarray:
    inv_scale = insz / outsz
    kernel_scale = max(inv_scale, 1.0)
    sample_f = (np.arange(outsz) + 0.5) * inv_scale - 0.5
    x = np.abs(sample_f[None, :] - np.arange(insz)[:, None]) / kernel_scale
    w = np.maximum(0.0, 1.0 - x)
    total = w.sum(axis=0, keepdims=True)
    w = np.where(np.abs(total) > 1000.0 * np.finfo(np.float32).eps,
                 w / np.where(total != 0, total, 1), 0.0)
    w = np.where(np.logical_and(sample_f >= -0.5, sample_f <= insz - 0.5)[None, :],
                 w, 0.0)
    return w.astype(np.float32)


_RESIZE_W = _resize_weight_mat(_SRC, _INSZ)


def _hist_loss_kernel(x_ref, w_ref, t_ref, out_ref,
                      p_s, q_s, r_s, iy_s, g_s, s_s, *, batch):
    b = pl.program_id(0)
    w = w_ref[...]

    zs = []
    logs = []
    for c in range(3):
        t = jnp.clip(x_ref[0, c], -1.0, 1.0)
        a = lax.dot_general(w, t, (((0,), (0,)), ((), ())),
                            preferred_element_type=jnp.float32)
        z = lax.dot_general(a, w, (((1,), (0,)), ((), ())),
                            preferred_element_type=jnp.float32)
        z = 0.5 * z + 0.5
        zs.append(z)
        logs.append(jnp.log2(z + _EPS))
    iy = jnp.sqrt(zs[0] * zs[0] + zs[1] * zs[1] + zs[2] * zs[2] + _EPS)

    inv_sig = jnp.float32(np.log(2.0) / 0.02)
    p_s[...] = ((logs[0] - logs[1]) * inv_sig).reshape(_N_CHUNK, _NP)
    q_s[...] = ((logs[0] - logs[2]) * inv_sig).reshape(_N_CHUNK, _NP)
    r_s[...] = ((logs[1] - logs[2]) * inv_sig).reshape(_N_CHUNK, _NP)
    iy_s[...] = iy.reshape(_N_CHUNK, _NP)

    delta = ((lax.broadcasted_iota(jnp.int32, (_HB, 1), 0).astype(jnp.float32)
              * (6.0 / 63.0) - 3.0) * jnp.float32(1.0 / 0.02))
    g_s[...] = jnp.zeros((2 * _HB, 2 * _HB), jnp.float32)

    def body(k, carry):
        pv = p_s[pl.ds(k, 1), :]
        qv = q_s[pl.ds(k, 1), :]
        rv = r_s[pl.ds(k, 1), :]
        wv = iy_s[pl.ds(k, 1), :]

        def kern(v):
            d = v - delta
            return pl.reciprocal(d * d + 1.0, approx=True)

        kp = kern(pv)
        kq = kern(qv)
        kr = kern(rv)
        bf = jnp.bfloat16
        w2 = jnp.concatenate([kp.astype(bf), kq.astype(bf)], axis=0)
        k2 = jnp.concatenate([(kq * wv).astype(bf), (kr * wv).astype(bf)],
                             axis=0)
        g_s[...] += lax.dot_general(w2, k2, (((1,), (1,)), ((), ())),
                                    preferred_element_type=jnp.float32)
        return carry

    lax.fori_loop(0, _N_CHUNK, body, 0, unroll=8)

    g = g_s[...]
    g3 = jnp.concatenate(
        [g[0:_HB, 0:_HB], g[0:_HB, _HB:2 * _HB], g[_HB:2 * _HB, _HB:2 * _HB]],
        axis=1)
    t3 = jnp.concatenate([t_ref[0], t_ref[1], t_ref[2]], axis=1)
    norm = jnp.sum(g3)
    s1 = jnp.sum(jnp.sqrt(g3 * t3))
    t0 = jnp.sum(t3)
    npe = norm + _EPS
    contrib = t0 + norm / npe - 2.0 * s1 * lax.rsqrt(npe)

    @pl.when(b == 0)
    def _():
        s_s[...] = jnp.zeros((1, 1), jnp.float32)

    s_s[...] += contrib.reshape(1, 1)

    @pl.when(b == batch - 1)
    def _():
        out_ref[...] = (jnp.float32(1.0 / np.sqrt(2.0)) / batch
                        ) * jnp.sqrt(s_s[...])


def kernel(rgbd, histogram_target):
    batch = rgbd.shape[0]
    t_arr = jnp.stack([
        histogram_target[0],
        histogram_target[1, ::-1, :],
        histogram_target[2, ::-1, ::-1],
    ])
    w = jnp.asarray(_RESIZE_W)

    out = pl.pallas_call(
        functools.partial(_hist_loss_kernel, batch=batch),
        grid=(batch,),
        in_specs=[
            pl.BlockSpec((1, 3, _SRC, _SRC), lambda b: (b, 0, 0, 0)),
            pl.BlockSpec((_SRC, _INSZ), lambda b: (0, 0)),
            pl.BlockSpec((3, _HB, _HB), lambda b: (0, 0, 0)),
        ],
        out_specs=pl.BlockSpec((1, 1), lambda b: (0, 0)),
        out_shape=jax.ShapeDtypeStruct((1, 1), jnp.float32),
        scratch_shapes=[
            pltpu.VMEM((_N_CHUNK, _NP), jnp.float32),
            pltpu.VMEM((_N_CHUNK, _NP), jnp.float32),
            pltpu.VMEM((_N_CHUNK, _NP), jnp.float32),
            pltpu.VMEM((_N_CHUNK, _NP), jnp.float32),
            pltpu.VMEM((2 * _HB, 2 * _HB), jnp.float32),
            pltpu.VMEM((1, 1), jnp.float32),
        ],
    )(rgbd, w, t_arr)
    return out[0, 0]


# speedup vs baseline: 1.0676x; 1.0676x over previous
import functools

import numpy as np
import jax
import jax.numpy as jnp
from jax import lax
from jax.experimental import pallas as pl
from jax.experimental.pallas import tpu as pltpu

_EPS = 1e-6
_HB = 64
_INSZ = 256
_SRC = 512
_R_CHUNK = 32
_N_CHUNK = _INSZ // _R_CHUNK
_NP = _R_CHUNK * _INSZ


def _resize_weight_mat(insz: int, outsz: int) -> np.ndarray:
    inv_scale = insz / outsz
    kernel_scale = max(inv_scale, 1.0)
    sample_f = (np.arange(outsz) + 0.5) * inv_scale - 0.5
    x = np.abs(sample_f[None, :] - np.arange(insz)[:, None]) / kernel_scale
    w = np.maximum(0.0, 1.0 - x)
    total = w.sum(axis=0, keepdims=True)
    w = np.where(np.abs(total) > 1000.0 * np.finfo(np.float32).eps,
                 w / np.where(total != 0, total, 1), 0.0)
    w = np.where(np.logical_and(sample_f >= -0.5, sample_f <= insz - 0.5)[None, :],
                 w, 0.0)
    return w.astype(np.float32)


_RESIZE_W = _resize_weight_mat(_SRC, _INSZ)


def _hist_loss_kernel(x_ref, w_ref, t_ref, out_ref,
                      p_s, q_s, iy_s, g_s, s_s, *, batch):
    b = pl.program_id(0)
    w = w_ref[...]

    zs = []
    logs = []
    for c in range(3):
        t = jnp.clip(x_ref[0, c], -1.0, 1.0)
        a = lax.dot_general(w, t, (((0,), (0,)), ((), ())),
                            preferred_element_type=jnp.float32)
        z = lax.dot_general(a, w, (((1,), (0,)), ((), ())),
                            preferred_element_type=jnp.float32)
        z = 0.5 * z + 0.5
        zs.append(z)
        logs.append(jnp.log2(z + _EPS))
    iy = jnp.sqrt(zs[0] * zs[0] + zs[1] * zs[1] + zs[2] * zs[2] + _EPS)

    inv_sig = jnp.float32(np.log(2.0) / 0.02)
    p_s[...] = ((logs[0] - logs[1]) * inv_sig).reshape(_N_CHUNK, _NP)
    q_s[...] = ((logs[0] - logs[2]) * inv_sig).reshape(_N_CHUNK, _NP)
    iy_s[...] = iy.reshape(_N_CHUNK, _NP)

    delta = ((lax.broadcasted_iota(jnp.int32, (_HB, 1), 0).astype(jnp.float32)
              * (6.0 / 63.0) - 3.0) * jnp.float32(1.0 / 0.02))
    g_s[...] = jnp.zeros((2 * _HB, 2 * _HB), jnp.float32)

    def body(k, carry):
        pv = p_s[pl.ds(k, 1), :]
        qv = q_s[pl.ds(k, 1), :]
        rv = qv - pv
        wv = iy_s[pl.ds(k, 1), :]

        def kern(v):
            d = v - delta
            return pl.reciprocal(d * d + 1.0, approx=True)

        kp = kern(pv)
        kq = kern(qv)
        kr = kern(rv)
        bf = jnp.bfloat16
        w2 = jnp.concatenate([kp.astype(bf), kq.astype(bf)], axis=0)
        k2 = jnp.concatenate([(kq * wv).astype(bf), (kr * wv).astype(bf)],
                             axis=0)
        g_s[...] += lax.dot_general(w2, k2, (((1,), (1,)), ((), ())),
                                    preferred_element_type=jnp.float32)
        return carry

    lax.fori_loop(0, _N_CHUNK, body, 0, unroll=4)

    g = g_s[...]
    g3 = jnp.concatenate(
        [g[0:_HB, 0:_HB], g[0:_HB, _HB:2 * _HB], g[_HB:2 * _HB, _HB:2 * _HB]],
        axis=1)
    t3 = jnp.concatenate([t_ref[0], t_ref[1], t_ref[2]], axis=1)
    norm = jnp.sum(g3)
    s1 = jnp.sum(jnp.sqrt(g3 * t3))
    t0 = jnp.sum(t3)
    npe = norm + _EPS
    contrib = t0 + norm / npe - 2.0 * s1 * lax.rsqrt(npe)

    @pl.when(b == 0)
    def _():
        s_s[...] = jnp.zeros((1, 1), jnp.float32)

    s_s[...] += contrib.reshape(1, 1)

    @pl.when(b == batch - 1)
    def _():
        out_ref[...] = (jnp.float32(1.0 / np.sqrt(2.0)) / batch
                        ) * jnp.sqrt(s_s[...])


def kernel(rgbd, histogram_target):
    batch = rgbd.shape[0]
    t_arr = jnp.stack([
        histogram_target[0],
        histogram_target[1, ::-1, :],
        histogram_target[2, ::-1, ::-1],
    ])
    w = jnp.asarray(_RESIZE_W)

    out = pl.pallas_call(
        functools.partial(_hist_loss_kernel, batch=batch),
        grid=(batch,),
        in_specs=[
            pl.BlockSpec((1, 3, _SRC, _SRC), lambda b: (b, 0, 0, 0)),
            pl.BlockSpec((_SRC, _INSZ), lambda b: (0, 0)),
            pl.BlockSpec((3, _HB, _HB), lambda b: (0, 0, 0)),
        ],
        out_specs=pl.BlockSpec((1, 1), lambda b: (0, 0)),
        out_shape=jax.ShapeDtypeStruct((1, 1), jnp.float32),
        scratch_shapes=[
            pltpu.VMEM((_N_CHUNK, _NP), jnp.float32),
            pltpu.VMEM((_N_CHUNK, _NP), jnp.float32),
            pltpu.VMEM((_N_CHUNK, _NP), jnp.float32),
            pltpu.VMEM((2 * _HB, 2 * _HB), jnp.float32),
            pltpu.VMEM((1, 1), jnp.float32),
        ],
    )(rgbd, w, t_arr)
    return out[0, 0]


# speedup vs baseline: 1.1143x; 1.0437x over previous
import functools

import numpy as np
import jax
import jax.numpy as jnp
from jax import lax
from jax.experimental import pallas as pl
from jax.experimental.pallas import tpu as pltpu

_EPS = 1e-6
_HB = 64
_INSZ = 256
_SRC = 512
_R_CHUNK = 64
_N_CHUNK = _INSZ // _R_CHUNK
_NP = _R_CHUNK * _INSZ


def _resize_weight_mat(insz: int, outsz: int) -> np.ndarray:
    inv_scale = insz / outsz
    kernel_scale = max(inv_scale, 1.0)
    sample_f = (np.arange(outsz) + 0.5) * inv_scale - 0.5
    x = np.abs(sample_f[None, :] - np.arange(insz)[:, None]) / kernel_scale
    w = np.maximum(0.0, 1.0 - x)
    total = w.sum(axis=0, keepdims=True)
    w = np.where(np.abs(total) > 1000.0 * np.finfo(np.float32).eps,
                 w / np.where(total != 0, total, 1), 0.0)
    w = np.where(np.logical_and(sample_f >= -0.5, sample_f <= insz - 0.5)[None, :],
                 w, 0.0)
    return w.astype(np.float32)


_RESIZE_W = _resize_weight_mat(_SRC, _INSZ)


def _hist_loss_kernel(x_ref, w_ref, t_ref, out_ref,
                      p_s, q_s, iy_s, g_s, s_s, *, batch):
    b = pl.program_id(0)
    w = w_ref[...]

    zs = []
    logs = []
    for c in range(3):
        t = jnp.clip(x_ref[0, c], -1.0, 1.0)
        a = lax.dot_general(w, t, (((0,), (0,)), ((), ())),
                            preferred_element_type=jnp.float32)
        z = lax.dot_general(a, w, (((1,), (0,)), ((), ())),
                            preferred_element_type=jnp.float32)
        z = 0.5 * z + 0.5
        zs.append(z)
        logs.append(jnp.log2(z + _EPS))
    iy = jnp.sqrt(zs[0] * zs[0] + zs[1] * zs[1] + zs[2] * zs[2] + _EPS)

    inv_sig = jnp.float32(np.log(2.0) / 0.02)
    p_s[...] = ((logs[0] - logs[1]) * inv_sig).reshape(_N_CHUNK, _NP)
    q_s[...] = ((logs[0] - logs[2]) * inv_sig).reshape(_N_CHUNK, _NP)
    iy_s[...] = iy.reshape(_N_CHUNK, _NP)

    delta = ((lax.broadcasted_iota(jnp.int32, (_HB, 1), 0).astype(jnp.float32)
              * (6.0 / 63.0) - 3.0) * jnp.float32(1.0 / 0.02))
    g_s[...] = jnp.zeros((2 * _HB, 2 * _HB), jnp.float32)

    def body(k, carry):
        pv = p_s[pl.ds(k, 1), :]
        qv = q_s[pl.ds(k, 1), :]
        rv = qv - pv
        wv = iy_s[pl.ds(k, 1), :]

        def kern(v):
            d = v - delta
            return pl.reciprocal(d * d + 1.0, approx=True)

        kp = kern(pv)
        kq = kern(qv)
        kr = kern(rv)
        bf = jnp.bfloat16
        w2 = jnp.concatenate([kp.astype(bf), kq.astype(bf)], axis=0)
        k2 = jnp.concatenate([(kq * wv).astype(bf), (kr * wv).astype(bf)],
                             axis=0)
        g_s[...] += lax.dot_general(w2, k2, (((1,), (1,)), ((), ())),
                                    preferred_element_type=jnp.float32)
        return carry

    lax.fori_loop(0, _N_CHUNK, body, 0, unroll=2)

    g = g_s[...]
    g3 = jnp.concatenate(
        [g[0:_HB, 0:_HB], g[0:_HB, _HB:2 * _HB], g[_HB:2 * _HB, _HB:2 * _HB]],
        axis=1)
    t3 = jnp.concatenate([t_ref[0], t_ref[1], t_ref[2]], axis=1)
    norm = jnp.sum(g3)
    s1 = jnp.sum(jnp.sqrt(g3 * t3))
    t0 = jnp.sum(t3)
    npe = norm + _EPS
    contrib = t0 + norm / npe - 2.0 * s1 * lax.rsqrt(npe)

    @pl.when(b == 0)
    def _():
        s_s[...] = jnp.zeros((1, 1), jnp.float32)

    s_s[...] += contrib.reshape(1, 1)

    @pl.when(b == batch - 1)
    def _():
        out_ref[...] = (jnp.float32(1.0 / np.sqrt(2.0)) / batch
                        ) * jnp.sqrt(s_s[...])


def kernel(rgbd, histogram_target):
    batch = rgbd.shape[0]
    t_arr = jnp.stack([
        histogram_target[0],
        histogram_target[1, ::-1, :],
        histogram_target[2, ::-1, ::-1],
    ])
    w = jnp.asarray(_RESIZE_W)

    out = pl.pallas_call(
        functools.partial(_hist_loss_kernel, batch=batch),
        grid=(batch,),
        in_specs=[
            pl.BlockSpec((1, 3, _SRC, _SRC), lambda b: (b, 0, 0, 0)),
            pl.BlockSpec((_SRC, _INSZ), lambda b: (0, 0)),
            pl.BlockSpec((3, _HB, _HB), lambda b: (0, 0, 0)),
        ],
        out_specs=pl.BlockSpec((1, 1), lambda b: (0, 0)),
        out_shape=jax.ShapeDtypeStruct((1, 1), jnp.float32),
        scratch_shapes=[
            pltpu.VMEM((_N_CHUNK, _NP), jnp.float32),
            pltpu.VMEM((_N_CHUNK, _NP), jnp.float32),
            pltpu.VMEM((_N_CHUNK, _NP), jnp.float32),
            pltpu.VMEM((2 * _HB, 2 * _HB), jnp.float32),
            pltpu.VMEM((1, 1), jnp.float32),
        ],
    )(rgbd, w, t_arr)
    return out[0, 0]


# speedup vs baseline: 1.1332x; 1.0170x over previous
import functools

import numpy as np
import jax
import jax.numpy as jnp
from jax import lax
from jax.experimental import pallas as pl
from jax.experimental.pallas import tpu as pltpu

_EPS = 1e-6
_HB = 64
_INSZ = 256
_SRC = 512
_R_CHUNK = 128
_N_CHUNK = _INSZ // _R_CHUNK
_NP = _R_CHUNK * _INSZ


def _resize_weight_mat(insz: int, outsz: int) -> np.ndarray:
    inv_scale = insz / outsz
    kernel_scale = max(inv_scale, 1.0)
    sample_f = (np.arange(outsz) + 0.5) * inv_scale - 0.5
    x = np.abs(sample_f[None, :] - np.arange(insz)[:, None]) / kernel_scale
    w = np.maximum(0.0, 1.0 - x)
    total = w.sum(axis=0, keepdims=True)
    w = np.where(np.abs(total) > 1000.0 * np.finfo(np.float32).eps,
                 w / np.where(total != 0, total, 1), 0.0)
    w = np.where(np.logical_and(sample_f >= -0.5, sample_f <= insz - 0.5)[None, :],
                 w, 0.0)
    return w.astype(np.float32)


_RESIZE_W = _resize_weight_mat(_SRC, _INSZ)


def _hist_loss_kernel(x_ref, w_ref, t_ref, out_ref,
                      p_s, q_s, iy_s, g_s, s_s, *, batch):
    b = pl.program_id(0)
    w = w_ref[...]

    zs = []
    logs = []
    for c in range(3):
        t = jnp.clip(x_ref[0, c], -1.0, 1.0)
        a = lax.dot_general(w, t, (((0,), (0,)), ((), ())),
                            preferred_element_type=jnp.float32)
        z = lax.dot_general(a, w, (((1,), (0,)), ((), ())),
                            preferred_element_type=jnp.float32)
        z = 0.5 * z + 0.5
        zs.append(z)
        logs.append(jnp.log2(z + _EPS))
    iy = jnp.sqrt(zs[0] * zs[0] + zs[1] * zs[1] + zs[2] * zs[2] + _EPS)

    inv_sig = jnp.float32(np.log(2.0) / 0.02)
    p_s[...] = ((logs[0] - logs[1]) * inv_sig).reshape(_N_CHUNK, _NP)
    q_s[...] = ((logs[0] - logs[2]) * inv_sig).reshape(_N_CHUNK, _NP)
    iy_s[...] = iy.reshape(_N_CHUNK, _NP)

    delta = ((lax.broadcasted_iota(jnp.int32, (_HB, 1), 0).astype(jnp.float32)
              * (6.0 / 63.0) - 3.0) * jnp.float32(1.0 / 0.02))
    g_s[...] = jnp.zeros((2 * _HB, 2 * _HB), jnp.float32)

    def body(k, carry):
        pv = p_s[pl.ds(k, 1), :]
        qv = q_s[pl.ds(k, 1), :]
        rv = qv - pv
        wv = iy_s[pl.ds(k, 1), :]

        def kern(v):
            d = v - delta
            return pl.reciprocal(d * d + 1.0, approx=True)

        kp = kern(pv)
        kq = kern(qv)
        kr = kern(rv)
        bf = jnp.bfloat16
        w2 = jnp.concatenate([kp.astype(bf), kq.astype(bf)], axis=0)
        k2 = jnp.concatenate([(kq * wv).astype(bf), (kr * wv).astype(bf)],
                             axis=0)
        g_s[...] += lax.dot_general(w2, k2, (((1,), (1,)), ((), ())),
                                    preferred_element_type=jnp.float32)
        return carry

    lax.fori_loop(0, _N_CHUNK, body, 0, unroll=2)

    g = g_s[...]
    g3 = jnp.concatenate(
        [g[0:_HB, 0:_HB], g[0:_HB, _HB:2 * _HB], g[_HB:2 * _HB, _HB:2 * _HB]],
        axis=1)
    t3 = jnp.concatenate([t_ref[0], t_ref[1], t_ref[2]], axis=1)
    norm = jnp.sum(g3)
    s1 = jnp.sum(jnp.sqrt(g3 * t3))
    t0 = jnp.sum(t3)
    npe = norm + _EPS
    contrib = t0 + norm / npe - 2.0 * s1 * lax.rsqrt(npe)

    @pl.when(b == 0)
    def _():
        s_s[...] = jnp.zeros((1, 1), jnp.float32)

    s_s[...] += contrib.reshape(1, 1)

    @pl.when(b == batch - 1)
    def _():
        out_ref[...] = (jnp.float32(1.0 / np.sqrt(2.0)) / batch
                        ) * jnp.sqrt(s_s[...])


def kernel(rgbd, histogram_target):
    batch = rgbd.shape[0]
    t_arr = jnp.stack([
        histogram_target[0],
        histogram_target[1, ::-1, :],
        histogram_target[2, ::-1, ::-1],
    ])
    w = jnp.asarray(_RESIZE_W)

    out = pl.pallas_call(
        functools.partial(_hist_loss_kernel, batch=batch),
        grid=(batch,),
        in_specs=[
            pl.BlockSpec((1, 3, _SRC, _SRC), lambda b: (b, 0, 0, 0)),
            pl.BlockSpec((_SRC, _INSZ), lambda b: (0, 0)),
            pl.BlockSpec((3, _HB, _HB), lambda b: (0, 0, 0)),
        ],
        out_specs=pl.BlockSpec((1, 1), lambda b: (0, 0)),
        out_shape=jax.ShapeDtypeStruct((1, 1), jnp.float32),
        scratch_shapes=[
            pltpu.VMEM((_N_CHUNK, _NP), jnp.float32),
            pltpu.VMEM((_N_CHUNK, _NP), jnp.float32),
            pltpu.VMEM((_N_CHUNK, _NP), jnp.float32),
            pltpu.VMEM((2 * _HB, 2 * _HB), jnp.float32),
            pltpu.VMEM((1, 1), jnp.float32),
        ],
    )(rgbd, w, t_arr)
    return out[0, 0]


# speedup vs baseline: 1.1369x; 1.0033x over previous
import functools

import numpy as np
import jax
import jax.numpy as jnp
from jax import lax
from jax.experimental import pallas as pl
from jax.experimental.pallas import tpu as pltpu

_EPS = 1e-6
_HB = 64
_INSZ = 256
_SRC = 512
_R_CHUNK = 256
_N_CHUNK = _INSZ // _R_CHUNK
_NP = _R_CHUNK * _INSZ


def _resize_weight_mat(insz: int, outsz: int) -> np.ndarray:
    inv_scale = insz / outsz
    kernel_scale = max(inv_scale, 1.0)
    sample_f = (np.arange(outsz) + 0.5) * inv_scale - 0.5
    x = np.abs(sample_f[None, :] - np.arange(insz)[:, None]) / kernel_scale
    w = np.maximum(0.0, 1.0 - x)
    total = w.sum(axis=0, keepdims=True)
    w = np.where(np.abs(total) > 1000.0 * np.finfo(np.float32).eps,
                 w / np.where(total != 0, total, 1), 0.0)
    w = np.where(np.logical_and(sample_f >= -0.5, sample_f <= insz - 0.5)[None, :],
                 w, 0.0)
    return w.astype(np.float32)


_RESIZE_W = _resize_weight_mat(_SRC, _INSZ)


def _hist_loss_kernel(x_ref, w_ref, t_ref, out_ref,
                      p_s, q_s, iy_s, g_s, s_s, *, batch):
    b = pl.program_id(0)
    w = w_ref[...]

    zs = []
    logs = []
    for c in range(3):
        t = jnp.clip(x_ref[0, c], -1.0, 1.0)
        a = lax.dot_general(w, t, (((0,), (0,)), ((), ())),
                            preferred_element_type=jnp.float32)
        z = lax.dot_general(a, w, (((1,), (0,)), ((), ())),
                            preferred_element_type=jnp.float32)
        z = 0.5 * z + 0.5
        zs.append(z)
        logs.append(jnp.log2(z + _EPS))
    iy = jnp.sqrt(zs[0] * zs[0] + zs[1] * zs[1] + zs[2] * zs[2] + _EPS)

    inv_sig = jnp.float32(np.log(2.0) / 0.02)
    p_s[...] = ((logs[0] - logs[1]) * inv_sig).reshape(_N_CHUNK, _NP)
    q_s[...] = ((logs[0] - logs[2]) * inv_sig).reshape(_N_CHUNK, _NP)
    iy_s[...] = iy.reshape(_N_CHUNK, _NP)

    delta = ((lax.broadcasted_iota(jnp.int32, (_HB, 1), 0).astype(jnp.float32)
              * (6.0 / 63.0) - 3.0) * jnp.float32(1.0 / 0.02))
    g_s[...] = jnp.zeros((2 * _HB, 2 * _HB), jnp.float32)

    def body(k, carry):
        pv = p_s[pl.ds(k, 1), :]
        qv = q_s[pl.ds(k, 1), :]
        rv = qv - pv
        wv = iy_s[pl.ds(k, 1), :]

        def kern(v):
            d = v - delta
            return pl.reciprocal(d * d + 1.0, approx=True)

        kp = kern(pv)
        kq = kern(qv)
        kr = kern(rv)
        bf = jnp.bfloat16
        w2 = jnp.concatenate([kp.astype(bf), kq.astype(bf)], axis=0)
        k2 = jnp.concatenate([(kq * wv).astype(bf), (kr * wv).astype(bf)],
                             axis=0)
        g_s[...] += lax.dot_general(w2, k2, (((1,), (1,)), ((), ())),
                                    preferred_element_type=jnp.float32)
        return carry

    lax.fori_loop(0, _N_CHUNK, body, 0, unroll=2)

    g = g_s[...]
    g3 = jnp.concatenate(
        [g[0:_HB, 0:_HB], g[0:_HB, _HB:2 * _HB], g[_HB:2 * _HB, _HB:2 * _HB]],
        axis=1)
    t3 = jnp.concatenate([t_ref[0], t_ref[1], t_ref[2]], axis=1)
    norm = jnp.sum(g3)
    s1 = jnp.sum(jnp.sqrt(g3 * t3))
    t0 = jnp.sum(t3)
    npe = norm + _EPS
    contrib = t0 + norm / npe - 2.0 * s1 * lax.rsqrt(npe)

    @pl.when(b == 0)
    def _():
        s_s[...] = jnp.zeros((1, 1), jnp.float32)

    s_s[...] += contrib.reshape(1, 1)

    @pl.when(b == batch - 1)
    def _():
        out_ref[...] = (jnp.float32(1.0 / np.sqrt(2.0)) / batch
                        ) * jnp.sqrt(s_s[...])


def kernel(rgbd, histogram_target):
    batch = rgbd.shape[0]
    t_arr = jnp.stack([
        histogram_target[0],
        histogram_target[1, ::-1, :],
        histogram_target[2, ::-1, ::-1],
    ])
    w = jnp.asarray(_RESIZE_W)

    out = pl.pallas_call(
        functools.partial(_hist_loss_kernel, batch=batch),
        grid=(batch,),
        in_specs=[
            pl.BlockSpec((1, 3, _SRC, _SRC), lambda b: (b, 0, 0, 0)),
            pl.BlockSpec((_SRC, _INSZ), lambda b: (0, 0)),
            pl.BlockSpec((3, _HB, _HB), lambda b: (0, 0, 0)),
        ],
        out_specs=pl.BlockSpec((1, 1), lambda b: (0, 0)),
        out_shape=jax.ShapeDtypeStruct((1, 1), jnp.float32),
        scratch_shapes=[
            pltpu.VMEM((_N_CHUNK, _NP), jnp.float32),
            pltpu.VMEM((_N_CHUNK, _NP), jnp.float32),
            pltpu.VMEM((_N_CHUNK, _NP), jnp.float32),
            pltpu.VMEM((2 * _HB, 2 * _HB), jnp.float32),
            pltpu.VMEM((1, 1), jnp.float32),
        ],
    )(rgbd, w, t_arr)
    return out[0, 0]


# speedup vs baseline: 1.1396x; 1.0024x over previous
import functools

import numpy as np
import jax
import jax.numpy as jnp
from jax import lax
from jax.experimental import pallas as pl
from jax.experimental.pallas import tpu as pltpu

_EPS = 1e-6
_HB = 64
_INSZ = 256
_SRC = 512
_R_CHUNK = 256
_N_CHUNK = _INSZ // _R_CHUNK
_NP = _R_CHUNK * _INSZ


def _resize_weight_mat(insz: int, outsz: int) -> np.ndarray:
    inv_scale = insz / outsz
    kernel_scale = max(inv_scale, 1.0)
    sample_f = (np.arange(outsz) + 0.5) * inv_scale - 0.5
    x = np.abs(sample_f[None, :] - np.arange(insz)[:, None]) / kernel_scale
    w = np.maximum(0.0, 1.0 - x)
    total = w.sum(axis=0, keepdims=True)
    w = np.where(np.abs(total) > 1000.0 * np.finfo(np.float32).eps,
                 w / np.where(total != 0, total, 1), 0.0)
    w = np.where(np.logical_and(sample_f >= -0.5, sample_f <= insz - 0.5)[None, :],
                 w, 0.0)
    return w.astype(np.float32)


_RESIZE_W = _resize_weight_mat(_SRC, _INSZ)


def _hist_loss_kernel(x_ref, w_ref, t_ref, out_ref,
                      p_s, q_s, iy_s, g_s, s_s, *, batch):
    b = pl.program_id(0)
    w = w_ref[...].astype(jnp.bfloat16)

    zs = []
    logs = []
    for c in range(3):
        t = jnp.clip(x_ref[0, c], -1.0, 1.0).astype(jnp.bfloat16)
        a = lax.dot_general(w, t, (((0,), (0,)), ((), ())),
                            preferred_element_type=jnp.float32)
        z = lax.dot_general(a.astype(jnp.bfloat16), w,
                            (((1,), (0,)), ((), ())),
                            preferred_element_type=jnp.float32)
        z = 0.5 * z + 0.5
        zs.append(z)
        logs.append(jnp.log2(z + _EPS))
    iy = jnp.sqrt(zs[0] * zs[0] + zs[1] * zs[1] + zs[2] * zs[2] + _EPS)

    inv_sig = jnp.float32(np.log(2.0) / 0.02)
    p_s[...] = ((logs[0] - logs[1]) * inv_sig).reshape(_N_CHUNK, _NP)
    q_s[...] = ((logs[0] - logs[2]) * inv_sig).reshape(_N_CHUNK, _NP)
    iy_s[...] = iy.reshape(_N_CHUNK, _NP)

    delta = ((lax.broadcasted_iota(jnp.int32, (_HB, 1), 0).astype(jnp.float32)
              * (6.0 / 63.0) - 3.0) * jnp.float32(1.0 / 0.02))
    g_s[...] = jnp.zeros((2 * _HB, 2 * _HB), jnp.float32)

    def body(k, carry):
        pv = p_s[pl.ds(k, 1), :]
        qv = q_s[pl.ds(k, 1), :]
        rv = qv - pv
        wv = iy_s[pl.ds(k, 1), :]

        def kern(v):
            d = v - delta
            return pl.reciprocal(d * d + 1.0, approx=True)

        kp = kern(pv)
        kq = kern(qv)
        kr = kern(rv)
        bf = jnp.bfloat16
        w2 = jnp.concatenate([kp.astype(bf), kq.astype(bf)], axis=0)
        k2 = jnp.concatenate([(kq * wv).astype(bf), (kr * wv).astype(bf)],
                             axis=0)
        g_s[...] += lax.dot_general(w2, k2, (((1,), (1,)), ((), ())),
                                    preferred_element_type=jnp.float32)
        return carry

    lax.fori_loop(0, _N_CHUNK, body, 0, unroll=2)

    g = g_s[...]
    g3 = jnp.concatenate(
        [g[0:_HB, 0:_HB], g[0:_HB, _HB:2 * _HB], g[_HB:2 * _HB, _HB:2 * _HB]],
        axis=1)
    t3 = jnp.concatenate([t_ref[0], t_ref[1], t_ref[2]], axis=1)
    norm = jnp.sum(g3)
    s1 = jnp.sum(jnp.sqrt(g3 * t3))
    t0 = jnp.sum(t3)
    npe = norm + _EPS
    contrib = t0 + norm / npe - 2.0 * s1 * lax.rsqrt(npe)

    @pl.when(b == 0)
    def _():
        s_s[...] = jnp.zeros((1, 1), jnp.float32)

    s_s[...] += contrib.reshape(1, 1)

    @pl.when(b == batch - 1)
    def _():
        out_ref[...] = (jnp.float32(1.0 / np.sqrt(2.0)) / batch
                        ) * jnp.sqrt(s_s[...])


def kernel(rgbd, histogram_target):
    batch = rgbd.shape[0]
    t_arr = jnp.stack([
        histogram_target[0],
        histogram_target[1, ::-1, :],
        histogram_target[2, ::-1, ::-1],
    ])
    w = jnp.asarray(_RESIZE_W)

    out = pl.pallas_call(
        functools.partial(_hist_loss_kernel, batch=batch),
        grid=(batch,),
        in_specs=[
            pl.BlockSpec((1, 3, _SRC, _SRC), lambda b: (b, 0, 0, 0)),
            pl.BlockSpec((_SRC, _INSZ), lambda b: (0, 0)),
            pl.BlockSpec((3, _HB, _HB), lambda b: (0, 0, 0)),
        ],
        out_specs=pl.BlockSpec((1, 1), lambda b: (0, 0)),
        out_shape=jax.ShapeDtypeStruct((1, 1), jnp.float32),
        scratch_shapes=[
            pltpu.VMEM((_N_CHUNK, _NP), jnp.float32),
            pltpu.VMEM((_N_CHUNK, _NP), jnp.float32),
            pltpu.VMEM((_N_CHUNK, _NP), jnp.float32),
            pltpu.VMEM((2 * _HB, 2 * _HB), jnp.float32),
            pltpu.VMEM((1, 1), jnp.float32),
        ],
    )(rgbd, w, t_arr)
    return out[0, 0]
